# baseline (speedup 1.0000x reference)
import functools

import jax
import jax.numpy as jnp
from jax import lax
from jax.experimental import pallas as pl
from jax.experimental.pallas import tpu as pltpu
from jax.experimental.pallas import tpu_sc as plsc

N = 10000
E = 320000
SD = 16
OC = 16
ED = 16
HID = 16
NPAD = 10240

NC = 2
NS = 16
NW = NC * NS
EPW = E // NW
CH = 2000
NCH = EPW // CH
RPT = NPAD // NS


CT = 16
EA_CH = 5
EA_HALF = CT * 1024


def _sc_gather(x, src, eav):
    mesh = plsc.VectorSubcoreMesh(core_axis_name="c", subcore_axis_name="s")

    @functools.partial(
        pl.kernel,
        out_type=(jax.ShapeDtypeStruct((E, SD), jnp.float32),
                  jax.ShapeDtypeStruct((E * ED,), jnp.float32)),
        mesh=mesh,
        scratch_types=[
            pltpu.VMEM((CH,), jnp.int32),
            pltpu.VMEM((CH, SD), jnp.float32),
            pltpu.SemaphoreType.DMA,
            pltpu.VMEM((2 * EA_HALF,), jnp.float32),
            pltpu.VMEM((2 * EA_HALF,), jnp.float32),
        ],
        compiler_params=pltpu.CompilerParams(use_tc_tiling_on_sc=False,
                                             needs_layout_passes=False),
    )
    def gather_k(x_hbm, src_hbm, eav_hbm, xj_hbm, eapk_hbm,
                 idx_v, rows_v, sem, ea_in, ea_out):
        cid = lax.axis_index("c")
        sid = lax.axis_index("s")
        wid = sid * NC + cid
        base = wid * EPW

        def body(j, carry):
            off = base + j * CH
            pltpu.sync_copy(src_hbm.at[pl.ds(off, CH)], idx_v)
            pltpu.async_copy(x_hbm.at[idx_v], rows_v, sem).wait()
            pltpu.sync_copy(rows_v, xj_hbm.at[pl.ds(off, CH)])
            return carry

        lax.fori_loop(0, NCH, body, 0)

        iota = lax.iota(jnp.int32, 16)
        fofs = (iota // 8) * EA_HALF + (iota % 8) * 128
        t0w = (wid * (2500 - EA_CH * CT)) // (NW - 1)

        def ea_body(ci, carry):
            t0 = t0w + ci * CT
            pltpu.sync_copy(eav_hbm.at[pl.ds(t0 * 1024, EA_HALF)],
                            ea_in.at[pl.ds(0, EA_HALF)])
            pltpu.sync_copy(eav_hbm.at[pl.ds(E * 8 + t0 * 1024, EA_HALF)],
                            ea_in.at[pl.ds(EA_HALF, EA_HALF)])

            def tl_body(tl, carry2):
                tbase = tl * 1024
                obase = tl * 2048

                def mm_body(mg, carry3):
                    for u in range(8):
                        v = plsc.load_gather(ea_in, [fofs + (tbase + mg * 8 + u)])
                        ea_out[pl.ds(obase + mg * 128 + u * 16, 16)] = v
                    return carry3

                lax.fori_loop(0, 16, mm_body, 0)
                return carry2

            lax.fori_loop(0, CT, tl_body, 0)
            pltpu.sync_copy(ea_out, eapk_hbm.at[pl.ds(t0 * 2048, CT * 2048)])
            return carry

        lax.fori_loop(0, EA_CH, ea_body, 0)

    return gather_k(x, src, eav)


def _tc_msg(ea_pk, xj_pk, W1, b1, W2, b2):
    E8 = E // 8
    TB = 1000
    grid = (E8 // TB,)
    ey8 = jnp.eye(8, dtype=jnp.float32)
    R = jnp.kron(jnp.eye(SD, dtype=jnp.float32), jnp.ones((1, OC), jnp.float32))
    S = jnp.kron(jnp.ones((SD, 1), jnp.float32), jnp.eye(OC, dtype=jnp.float32))
    W1b = jnp.kron(ey8, W1)
    b1b = jnp.tile(b1, 8).reshape(1, 128)
    W2b = jnp.kron(ey8, W2)
    b2b = jnp.tile(b2, 8).reshape(1, 8 * SD * OC)
    Rb = jnp.kron(ey8, R)
    Sb = jnp.kron(ey8, S)

    def body(ea_ref, xj_ref, w1_ref, b1_ref, w2_ref, b2_ref, r_ref, s_ref,
             out_ref):
        h = jnp.maximum(
            jnp.dot(ea_ref[...], w1_ref[...], preferred_element_type=jnp.float32)
            + b1_ref[...], 0.0)
        w = jnp.dot(h, w2_ref[...], preferred_element_type=jnp.float32) + b2_ref[...]
        xr = jnp.dot(xj_ref[...], r_ref[...], preferred_element_type=jnp.float32)
        out_ref[...] = jnp.dot(xr * w, s_ref[...],
                               preferred_element_type=jnp.float32)

    C = 8 * SD * OC
    return pl.pallas_call(
        body,
        grid=grid,
        in_specs=[
            pl.BlockSpec((TB, 128), lambda i: (i, 0)),
            pl.BlockSpec((TB, 128), lambda i: (i, 0)),
            pl.BlockSpec((128, 128), lambda i: (0, 0)),
            pl.BlockSpec((1, 128), lambda i: (0, 0)),
            pl.BlockSpec((128, C), lambda i: (0, 0)),
            pl.BlockSpec((1, C), lambda i: (0, 0)),
            pl.BlockSpec((128, C), lambda i: (0, 0)),
            pl.BlockSpec((C, 128), lambda i: (0, 0)),
        ],
        out_specs=pl.BlockSpec((TB, 128), lambda i: (i, 0)),
        out_shape=jax.ShapeDtypeStruct((E8, 128), jnp.float32),
    )(ea_pk, xj_pk, W1b, b1b, W2b, b2b, Rb, Sb)


def _sc_scatter(msg, dst, z2d, z1d, ones_c):
    mesh = plsc.VectorSubcoreMesh(core_axis_name="c", subcore_axis_name="s")

    @functools.partial(
        pl.kernel,
        out_type=(jax.ShapeDtypeStruct((NC * NPAD, OC), jnp.float32),
                  jax.ShapeDtypeStruct((NC * NPAD,), jnp.float32)),
        mesh=mesh,
        scratch_types=[
            pltpu.VMEM((CH,), jnp.int32),
            pltpu.VMEM((CH, OC), jnp.float32),
            pltpu.VMEM((CH,), jnp.float32),
            pltpu.VMEM_SHARED((NPAD, OC), jnp.float32),
            pltpu.VMEM_SHARED((NPAD,), jnp.float32),
        ],
        compiler_params=pltpu.CompilerParams(use_tc_tiling_on_sc=False),
    )
    def scatter_k(msg_hbm, dst_hbm, z2d_hbm, z1d_hbm, ones_hbm,
                  agg_hbm, cnt_hbm, idx_v, msg_v, ones_v, acc_sh, cnt_sh):
        cid = lax.axis_index("c")
        sid = lax.axis_index("s")

        @pl.when(sid == 0)
        def _zero():
            pltpu.sync_copy(z2d_hbm, acc_sh)
            pltpu.sync_copy(z1d_hbm, cnt_sh)

        pltpu.sync_copy(ones_hbm, ones_v)
        plsc.subcore_barrier()

        base = (sid * NC + cid) * EPW

        def body(j, carry):
            off = base + j * CH
            pltpu.sync_copy(dst_hbm.at[pl.ds(off, CH)], idx_v)
            pltpu.sync_copy(msg_hbm.at[pl.ds(off, CH)], msg_v)
            pltpu.sync_copy(msg_v, acc_sh.at[idx_v], add=True)
            pltpu.sync_copy(ones_v, cnt_sh.at[idx_v], add=True)
            return carry

        lax.fori_loop(0, NCH, body, 0)
        plsc.subcore_barrier()

        ro = sid * RPT
        pltpu.sync_copy(acc_sh.at[pl.ds(ro, RPT)],
                        agg_hbm.at[pl.ds(cid * NPAD + ro, RPT)])
        pltpu.sync_copy(cnt_sh.at[pl.ds(ro, RPT)],
                        cnt_hbm.at[pl.ds(cid * NPAD + ro, RPT)])

    return scatter_k(msg, dst, z2d, z1d, ones_c)


def _tc_final(parts, cnts, xpad, maskf, act, root, bias,
              M1a, M1b, mb1, M2, mb2, M3, mb3):
    T = 1280
    G = NPAD // T

    def body(p0, p1, c0, c1, xr, mr, ar, root_r, bias_r,
             m1a, m1b, mb1_r, m2, mb2_r, m3, mb3_r, yr):
        cnt = jnp.maximum(c0[...] + c1[...], 1.0)
        agg = (p0[...] + p1[...]) / cnt
        out = agg + jnp.dot(xr[...], root_r[...],
                            preferred_element_type=jnp.float32) + bias_r[...]
        out = out * mr[...]
        v1 = (jnp.dot(out, m1a[...], preferred_element_type=jnp.float32)
              + ar[...] * m1b[...] + mb1_r[...])
        z1 = jnp.where(v1 > 0, v1, jnp.exp(jnp.minimum(v1, 0.0)) - 1.0)
        v2 = jnp.dot(z1, m2[...], preferred_element_type=jnp.float32) + mb2_r[...]
        z2 = jnp.where(v2 > 0, v2, jnp.exp(jnp.minimum(v2, 0.0)) - 1.0)
        yr[...] = jnp.dot(z2, m3[...], preferred_element_type=jnp.float32) + mb3_r[...]

    return pl.pallas_call(
        body,
        grid=(G,),
        in_specs=[
            pl.BlockSpec((T, OC), lambda i: (i, 0)),
            pl.BlockSpec((T, OC), lambda i: (i + G, 0)),
            pl.BlockSpec((T, 1), lambda i: (i, 0)),
            pl.BlockSpec((T, 1), lambda i: (i + G, 0)),
            pl.BlockSpec((T, SD), lambda i: (i, 0)),
            pl.BlockSpec((T, 1), lambda i: (i, 0)),
            pl.BlockSpec((T, 1), lambda i: (i, 0)),
            pl.BlockSpec((SD, OC), lambda i: (0, 0)),
            pl.BlockSpec((1, OC), lambda i: (0, 0)),
            pl.BlockSpec((OC, 64), lambda i: (0, 0)),
            pl.BlockSpec((1, 64), lambda i: (0, 0)),
            pl.BlockSpec((1, 64), lambda i: (0, 0)),
            pl.BlockSpec((64, 64), lambda i: (0, 0)),
            pl.BlockSpec((1, 64), lambda i: (0, 0)),
            pl.BlockSpec((64, 1), lambda i: (0, 0)),
            pl.BlockSpec((1, 1), lambda i: (0, 0)),
        ],
        out_specs=pl.BlockSpec((T, 1), lambda i: (i, 0)),
        out_shape=jax.ShapeDtypeStruct((NPAD, 1), jnp.float32),
    )(parts, parts, cnts, cnts, xpad, maskf, act, root, bias,
      M1a, M1b, mb1, M2, mb2, M3, mb3)


def kernel(x, edge_index, edge_attr, mask, batch, action,
           W1, b1, W2, b2, root, bias, M1, mb1, M2, mb2, M3, mb3):
    src = edge_index[0].astype(jnp.int32)
    dst = edge_index[1].astype(jnp.int32)

    eav = (edge_attr.T.reshape(2, 8, 2500, 128)
           .transpose(0, 2, 1, 3).reshape(E * ED))
    xj, eapk_flat = _sc_gather(x, src, eav)
    ea_pk = eapk_flat.reshape(E // 8, 8 * ED)
    xj_pk = xj.reshape(E // 8, 8 * SD)
    msg_pk = _tc_msg(ea_pk, xj_pk, W1, b1, W2, b2)
    msg = msg_pk.reshape(E, OC)

    z2d = jnp.zeros((NPAD, OC), jnp.float32)
    z1d = jnp.zeros((NPAD,), jnp.float32)
    ones_c = jnp.ones((CH,), jnp.float32)
    parts, cnts = _sc_scatter(msg, dst, z2d, z1d, ones_c)

    pad = NPAD - N
    xpad = jnp.pad(x, ((0, pad), (0, 0)))
    maskf = jnp.pad(mask.astype(jnp.float32), (0, pad)).reshape(NPAD, 1)
    act = jnp.pad(action.astype(jnp.float32), (0, pad)).reshape(NPAD, 1)

    y = _tc_final(parts, cnts.reshape(NC * NPAD, 1), xpad, maskf, act,
                  root, bias.reshape(1, OC),
                  M1[:OC], M1[OC:OC + 1], mb1.reshape(1, 64),
                  M2, mb2.reshape(1, 64), M3, mb3.reshape(1, 1))
    return y[:N]


# speedup vs baseline: 1.1442x; 1.1442x over previous
import functools

import jax
import jax.numpy as jnp
from jax import lax
from jax.experimental import pallas as pl
from jax.experimental.pallas import tpu as pltpu
from jax.experimental.pallas import tpu_sc as plsc

N = 10000
E = 320000
SD = 16
OC = 16
ED = 16
HID = 16
NPAD = 10240

NC = 2
NS = 16
NW = NC * NS
EPW = E // NW
CH = 2000
NCH = EPW // CH
RPT = NPAD // NS


CT = 16
EA_CH = 5
EA_HALF = CT * 1024


def _sc_gather(x, src, eav):
    mesh = plsc.VectorSubcoreMesh(core_axis_name="c", subcore_axis_name="s")

    @functools.partial(
        pl.kernel,
        out_type=(jax.ShapeDtypeStruct((E, SD), jnp.float32),
                  jax.ShapeDtypeStruct((E * ED,), jnp.float32)),
        mesh=mesh,
        scratch_types=[
            pltpu.VMEM((CH,), jnp.int32),
            pltpu.VMEM((CH, SD), jnp.float32),
            pltpu.SemaphoreType.DMA,
            pltpu.VMEM((2 * EA_HALF,), jnp.float32),
            pltpu.VMEM((2 * EA_HALF,), jnp.float32),
        ],
        compiler_params=pltpu.CompilerParams(use_tc_tiling_on_sc=False,
                                             needs_layout_passes=False),
    )
    def gather_k(x_hbm, src_hbm, eav_hbm, xj_hbm, eapk_hbm,
                 idx_v, rows_v, sem, ea_in, ea_out):
        cid = lax.axis_index("c")
        sid = lax.axis_index("s")
        wid = sid * NC + cid
        base = wid * EPW

        def body(j, carry):
            off = base + j * CH
            pltpu.sync_copy(src_hbm.at[pl.ds(off, CH)], idx_v)
            pltpu.async_copy(x_hbm.at[idx_v], rows_v, sem).wait()
            pltpu.sync_copy(rows_v, xj_hbm.at[pl.ds(off, CH)])
            return carry

        lax.fori_loop(0, NCH, body, 0)

        iota = lax.iota(jnp.int32, 16)
        fofs = (iota // 8) * EA_HALF + (iota % 8) * 128
        t0w = (wid * (2500 - EA_CH * CT)) // (NW - 1)

        def ea_body(ci, carry):
            t0 = t0w + ci * CT
            pltpu.sync_copy(eav_hbm.at[pl.ds(t0 * 1024, EA_HALF)],
                            ea_in.at[pl.ds(0, EA_HALF)])
            pltpu.sync_copy(eav_hbm.at[pl.ds(E * 8 + t0 * 1024, EA_HALF)],
                            ea_in.at[pl.ds(EA_HALF, EA_HALF)])

            @plsc.parallel_loop(0, CT * 128, step=8)
            def _transpose(m0):
                tl = m0 // 128
                mm0 = m0 - tl * 128
                bvec = fofs + (tl * 1024 + mm0)
                obase = tl * 2048 + mm0 * 16
                for u in range(8):
                    v = plsc.load_gather(ea_in, [bvec + u])
                    ea_out[pl.ds(pl.multiple_of(obase + u * 16, 16), 16)] = v
            pltpu.sync_copy(ea_out, eapk_hbm.at[pl.ds(t0 * 2048, CT * 2048)])
            return carry

        lax.fori_loop(0, EA_CH, ea_body, 0)

    return gather_k(x, src, eav)


def _tc_msg(ea_pk, xj_pk, W1, b1, W2, b2):
    E8 = E // 8
    TB = 1000
    grid = (E8 // TB,)
    ey8 = jnp.eye(8, dtype=jnp.float32)
    R = jnp.kron(jnp.eye(SD, dtype=jnp.float32), jnp.ones((1, OC), jnp.float32))
    S = jnp.kron(jnp.ones((SD, 1), jnp.float32), jnp.eye(OC, dtype=jnp.float32))
    W1b = jnp.kron(ey8, W1)
    b1b = jnp.tile(b1, 8).reshape(1, 128)
    W2b = jnp.kron(ey8, W2)
    b2b = jnp.tile(b2, 8).reshape(1, 8 * SD * OC)
    Rb = jnp.kron(ey8, R)
    Sb = jnp.kron(ey8, S)

    def body(ea_ref, xj_ref, w1_ref, b1_ref, w2_ref, b2_ref, r_ref, s_ref,
             out_ref):
        h = jnp.maximum(
            jnp.dot(ea_ref[...], w1_ref[...], preferred_element_type=jnp.float32)
            + b1_ref[...], 0.0)
        w = jnp.dot(h, w2_ref[...], preferred_element_type=jnp.float32) + b2_ref[...]
        xr = jnp.dot(xj_ref[...], r_ref[...], preferred_element_type=jnp.float32)
        out_ref[...] = jnp.dot(xr * w, s_ref[...],
                               preferred_element_type=jnp.float32)

    C = 8 * SD * OC
    return pl.pallas_call(
        body,
        grid=grid,
        in_specs=[
            pl.BlockSpec((TB, 128), lambda i: (i, 0)),
            pl.BlockSpec((TB, 128), lambda i: (i, 0)),
            pl.BlockSpec((128, 128), lambda i: (0, 0)),
            pl.BlockSpec((1, 128), lambda i: (0, 0)),
            pl.BlockSpec((128, C), lambda i: (0, 0)),
            pl.BlockSpec((1, C), lambda i: (0, 0)),
            pl.BlockSpec((128, C), lambda i: (0, 0)),
            pl.BlockSpec((C, 128), lambda i: (0, 0)),
        ],
        out_specs=pl.BlockSpec((TB, 128), lambda i: (i, 0)),
        out_shape=jax.ShapeDtypeStruct((E8, 128), jnp.float32),
    )(ea_pk, xj_pk, W1b, b1b, W2b, b2b, Rb, Sb)


def _sc_scatter(msg, dst, z2d, z1d, ones_c):
    mesh = plsc.VectorSubcoreMesh(core_axis_name="c", subcore_axis_name="s")

    @functools.partial(
        pl.kernel,
        out_type=(jax.ShapeDtypeStruct((NC * NPAD, OC), jnp.float32),
                  jax.ShapeDtypeStruct((NC * NPAD,), jnp.float32)),
        mesh=mesh,
        scratch_types=[
            pltpu.VMEM((CH,), jnp.int32),
            pltpu.VMEM((CH, OC), jnp.float32),
            pltpu.VMEM((CH,), jnp.float32),
            pltpu.VMEM_SHARED((NPAD, OC), jnp.float32),
            pltpu.VMEM_SHARED((NPAD,), jnp.float32),
        ],
        compiler_params=pltpu.CompilerParams(use_tc_tiling_on_sc=False),
    )
    def scatter_k(msg_hbm, dst_hbm, z2d_hbm, z1d_hbm, ones_hbm,
                  agg_hbm, cnt_hbm, idx_v, msg_v, ones_v, acc_sh, cnt_sh):
        cid = lax.axis_index("c")
        sid = lax.axis_index("s")

        @pl.when(sid == 0)
        def _zero():
            pltpu.sync_copy(z2d_hbm, acc_sh)
            pltpu.sync_copy(z1d_hbm, cnt_sh)

        pltpu.sync_copy(ones_hbm, ones_v)
        plsc.subcore_barrier()

        base = (sid * NC + cid) * EPW

        def body(j, carry):
            off = base + j * CH
            pltpu.sync_copy(dst_hbm.at[pl.ds(off, CH)], idx_v)
            pltpu.sync_copy(msg_hbm.at[pl.ds(off, CH)], msg_v)
            pltpu.sync_copy(msg_v, acc_sh.at[idx_v], add=True)
            pltpu.sync_copy(ones_v, cnt_sh.at[idx_v], add=True)
            return carry

        lax.fori_loop(0, NCH, body, 0)
        plsc.subcore_barrier()

        ro = sid * RPT
        pltpu.sync_copy(acc_sh.at[pl.ds(ro, RPT)],
                        agg_hbm.at[pl.ds(cid * NPAD + ro, RPT)])
        pltpu.sync_copy(cnt_sh.at[pl.ds(ro, RPT)],
                        cnt_hbm.at[pl.ds(cid * NPAD + ro, RPT)])

    return scatter_k(msg, dst, z2d, z1d, ones_c)


def _tc_final(parts, cnts, xpad, maskf, act, root, bias,
              M1a, M1b, mb1, M2, mb2, M3, mb3):
    T = 1280
    G = NPAD // T

    def body(p0, p1, c0, c1, xr, mr, ar, root_r, bias_r,
             m1a, m1b, mb1_r, m2, mb2_r, m3, mb3_r, yr):
        cnt = jnp.maximum(c0[...] + c1[...], 1.0)
        agg = (p0[...] + p1[...]) / cnt
        out = agg + jnp.dot(xr[...], root_r[...],
                            preferred_element_type=jnp.float32) + bias_r[...]
        out = out * mr[...]
        v1 = (jnp.dot(out, m1a[...], preferred_element_type=jnp.float32)
              + ar[...] * m1b[...] + mb1_r[...])
        z1 = jnp.where(v1 > 0, v1, jnp.exp(jnp.minimum(v1, 0.0)) - 1.0)
        v2 = jnp.dot(z1, m2[...], preferred_element_type=jnp.float32) + mb2_r[...]
        z2 = jnp.where(v2 > 0, v2, jnp.exp(jnp.minimum(v2, 0.0)) - 1.0)
        yr[...] = jnp.dot(z2, m3[...], preferred_element_type=jnp.float32) + mb3_r[...]

    return pl.pallas_call(
        body,
        grid=(G,),
        in_specs=[
            pl.BlockSpec((T, OC), lambda i: (i, 0)),
            pl.BlockSpec((T, OC), lambda i: (i + G, 0)),
            pl.BlockSpec((T, 1), lambda i: (i, 0)),
            pl.BlockSpec((T, 1), lambda i: (i + G, 0)),
            pl.BlockSpec((T, SD), lambda i: (i, 0)),
            pl.BlockSpec((T, 1), lambda i: (i, 0)),
            pl.BlockSpec((T, 1), lambda i: (i, 0)),
            pl.BlockSpec((SD, OC), lambda i: (0, 0)),
            pl.BlockSpec((1, OC), lambda i: (0, 0)),
            pl.BlockSpec((OC, 64), lambda i: (0, 0)),
            pl.BlockSpec((1, 64), lambda i: (0, 0)),
            pl.BlockSpec((1, 64), lambda i: (0, 0)),
            pl.BlockSpec((64, 64), lambda i: (0, 0)),
            pl.BlockSpec((1, 64), lambda i: (0, 0)),
            pl.BlockSpec((64, 1), lambda i: (0, 0)),
            pl.BlockSpec((1, 1), lambda i: (0, 0)),
        ],
        out_specs=pl.BlockSpec((T, 1), lambda i: (i, 0)),
        out_shape=jax.ShapeDtypeStruct((NPAD, 1), jnp.float32),
    )(parts, parts, cnts, cnts, xpad, maskf, act, root, bias,
      M1a, M1b, mb1, M2, mb2, M3, mb3)


def kernel(x, edge_index, edge_attr, mask, batch, action,
           W1, b1, W2, b2, root, bias, M1, mb1, M2, mb2, M3, mb3):
    src = edge_index[0].astype(jnp.int32)
    dst = edge_index[1].astype(jnp.int32)

    eav = (edge_attr.T.reshape(2, 8, 2500, 128)
           .transpose(0, 2, 1, 3).reshape(E * ED))
    xj, eapk_flat = _sc_gather(x, src, eav)
    ea_pk = eapk_flat.reshape(E // 8, 8 * ED)
    xj_pk = xj.reshape(E // 8, 8 * SD)
    msg_pk = _tc_msg(ea_pk, xj_pk, W1, b1, W2, b2)
    msg = msg_pk.reshape(E, OC)

    z2d = jnp.zeros((NPAD, OC), jnp.float32)
    z1d = jnp.zeros((NPAD,), jnp.float32)
    ones_c = jnp.ones((CH,), jnp.float32)
    parts, cnts = _sc_scatter(msg, dst, z2d, z1d, ones_c)

    pad = NPAD - N
    xpad = jnp.pad(x, ((0, pad), (0, 0)))
    maskf = jnp.pad(mask.astype(jnp.float32), (0, pad)).reshape(NPAD, 1)
    act = jnp.pad(action.astype(jnp.float32), (0, pad)).reshape(NPAD, 1)

    y = _tc_final(parts, cnts.reshape(NC * NPAD, 1), xpad, maskf, act,
                  root, bias.reshape(1, OC),
                  M1[:OC], M1[OC:OC + 1], mb1.reshape(1, 64),
                  M2, mb2.reshape(1, 64), M3, mb3.reshape(1, 1))
    return y[:N]


# speedup vs baseline: 1.2953x; 1.1321x over previous
import functools

import jax
import jax.numpy as jnp
from jax import lax
from jax.experimental import pallas as pl
from jax.experimental.pallas import tpu as pltpu
from jax.experimental.pallas import tpu_sc as plsc

N = 10000
E = 320000
EH = E // 2
SD = 16
OC = 16
ED = 16
HID = 16
NPAD = 10240

NC = 2
NS = 16
NW = NC * NS
EPW = EH // NW
CH = 1000
NCH = EPW // CH
RPT = NPAD // NS

CT = 8
EA_CH = 5
EA_HALF = CT * 1024
THALF = EH // 128


def _sc_front(x, src, eav, half):
    mesh = plsc.VectorSubcoreMesh(core_axis_name="c", subcore_axis_name="s")

    @functools.partial(
        pl.kernel,
        out_type=(jax.ShapeDtypeStruct((EH, SD), jnp.float32),
                  jax.ShapeDtypeStruct((EH * ED,), jnp.float32)),
        mesh=mesh,
        scratch_types=[
            pltpu.VMEM((CH,), jnp.int32),
            pltpu.VMEM((CH, SD), jnp.float32),
            pltpu.SemaphoreType.DMA,
            pltpu.VMEM((2 * EA_HALF,), jnp.float32),
            pltpu.VMEM((2 * EA_HALF,), jnp.float32),
        ],
        compiler_params=pltpu.CompilerParams(use_tc_tiling_on_sc=False,
                                             needs_layout_passes=False),
    )
    def front_k(x_hbm, src_hbm, eav_hbm, xj_hbm, eapk_hbm,
                idx_v, rows_v, sem, ea_in, ea_out):
        cid = lax.axis_index("c")
        sid = lax.axis_index("s")
        wid = sid * NC + cid
        base = wid * EPW

        def body(j, carry):
            off = base + j * CH
            pltpu.sync_copy(src_hbm.at[pl.ds(half * EH + off, CH)], idx_v)
            pltpu.async_copy(x_hbm.at[idx_v], rows_v, sem).wait()
            pltpu.sync_copy(rows_v, xj_hbm.at[pl.ds(off, CH)])
            return carry

        lax.fori_loop(0, NCH, body, 0)

        iota = lax.iota(jnp.int32, 16)
        fofs = (iota // 8) * EA_HALF + (iota % 8) * 128
        t0w = (wid * (THALF - EA_CH * CT)) // (NW - 1)

        def ea_body(ci, carry):
            t0 = t0w + ci * CT
            tg = half * THALF + t0
            pltpu.sync_copy(eav_hbm.at[pl.ds(tg * 1024, EA_HALF)],
                            ea_in.at[pl.ds(0, EA_HALF)])
            pltpu.sync_copy(eav_hbm.at[pl.ds(E * 8 + tg * 1024, EA_HALF)],
                            ea_in.at[pl.ds(EA_HALF, EA_HALF)])

            @plsc.parallel_loop(0, CT * 128, step=8)
            def _transpose(m0):
                tl = m0 // 128
                mm0 = m0 - tl * 128
                bvec = fofs + (tl * 1024 + mm0)
                obase = tl * 2048 + mm0 * 16
                for u in range(8):
                    v = plsc.load_gather(ea_in, [bvec + u])
                    ea_out[pl.ds(pl.multiple_of(obase + u * 16, 16), 16)] = v

            pltpu.sync_copy(ea_out, eapk_hbm.at[pl.ds(t0 * 2048, CT * 2048)])
            return carry

        lax.fori_loop(0, EA_CH, ea_body, 0)

    return front_k(x, src, eav)


def _tc_msg(ea_pk, xj_pk, W1, b1, W2, b2):
    E8 = EH // 8
    TB = 1000
    grid = (E8 // TB,)
    ey8 = jnp.eye(8, dtype=jnp.float32)
    R = jnp.kron(jnp.eye(SD, dtype=jnp.float32), jnp.ones((1, OC), jnp.float32))
    S = jnp.kron(jnp.ones((SD, 1), jnp.float32), jnp.eye(OC, dtype=jnp.float32))
    W1b = jnp.kron(ey8, W1)
    b1b = jnp.tile(b1, 8).reshape(1, 128)
    W2b = jnp.kron(ey8, W2)
    b2b = jnp.tile(b2, 8).reshape(1, 8 * SD * OC)
    Rb = jnp.kron(ey8, R)
    Sb = jnp.kron(ey8, S)

    def body(ea_ref, xj_ref, w1_ref, b1_ref, w2_ref, b2_ref, r_ref, s_ref,
             out_ref):
        h = jnp.maximum(
            jnp.dot(ea_ref[...], w1_ref[...], preferred_element_type=jnp.float32)
            + b1_ref[...], 0.0)
        w = jnp.dot(h, w2_ref[...], preferred_element_type=jnp.float32) + b2_ref[...]
        xr = jnp.dot(xj_ref[...], r_ref[...], preferred_element_type=jnp.float32)
        out_ref[...] = jnp.dot(xr * w, s_ref[...],
                               preferred_element_type=jnp.float32)

    C = 8 * SD * OC
    return pl.pallas_call(
        body,
        grid=grid,
        in_specs=[
            pl.BlockSpec((TB, 128), lambda i: (i, 0)),
            pl.BlockSpec((TB, 128), lambda i: (i, 0)),
            pl.BlockSpec((128, 128), lambda i: (0, 0)),
            pl.BlockSpec((1, 128), lambda i: (0, 0)),
            pl.BlockSpec((128, C), lambda i: (0, 0)),
            pl.BlockSpec((1, C), lambda i: (0, 0)),
            pl.BlockSpec((128, C), lambda i: (0, 0)),
            pl.BlockSpec((C, 128), lambda i: (0, 0)),
        ],
        out_specs=pl.BlockSpec((TB, 128), lambda i: (i, 0)),
        out_shape=jax.ShapeDtypeStruct((E8, 128), jnp.float32),
    )(ea_pk, xj_pk, W1b, b1b, W2b, b2b, Rb, Sb)


def _sc_scatter(msg, dst, z2d, z1d, ones_c, half):
    mesh = plsc.VectorSubcoreMesh(core_axis_name="c", subcore_axis_name="s")

    @functools.partial(
        pl.kernel,
        out_type=(jax.ShapeDtypeStruct((NC * NPAD, OC), jnp.float32),
                  jax.ShapeDtypeStruct((NC * NPAD,), jnp.float32)),
        mesh=mesh,
        scratch_types=[
            pltpu.VMEM((CH,), jnp.int32),
            pltpu.VMEM((CH, OC), jnp.float32),
            pltpu.VMEM((CH,), jnp.float32),
            pltpu.VMEM_SHARED((NPAD, OC), jnp.float32),
            pltpu.VMEM_SHARED((NPAD,), jnp.float32),
        ],
        compiler_params=pltpu.CompilerParams(use_tc_tiling_on_sc=False),
    )
    def scatter_k(msg_hbm, dst_hbm, z2d_hbm, z1d_hbm, ones_hbm,
                  agg_hbm, cnt_hbm, idx_v, msg_v, ones_v, acc_sh, cnt_sh):
        cid = lax.axis_index("c")
        sid = lax.axis_index("s")

        @pl.when(sid == 0)
        def _zero():
            pltpu.sync_copy(z2d_hbm, acc_sh)
            pltpu.sync_copy(z1d_hbm, cnt_sh)

        pltpu.sync_copy(ones_hbm, ones_v)
        plsc.subcore_barrier()

        base = (sid * NC + cid) * EPW

        def body(j, carry):
            off = base + j * CH
            pltpu.sync_copy(dst_hbm.at[pl.ds(half * EH + off, CH)], idx_v)
            pltpu.sync_copy(msg_hbm.at[pl.ds(off, CH)], msg_v)
            pltpu.sync_copy(msg_v, acc_sh.at[idx_v], add=True)
            pltpu.sync_copy(ones_v, cnt_sh.at[idx_v], add=True)
            return carry

        lax.fori_loop(0, NCH, body, 0)
        plsc.subcore_barrier()

        ro = sid * RPT
        pltpu.sync_copy(acc_sh.at[pl.ds(ro, RPT)],
                        agg_hbm.at[pl.ds(cid * NPAD + ro, RPT)])
        pltpu.sync_copy(cnt_sh.at[pl.ds(ro, RPT)],
                        cnt_hbm.at[pl.ds(cid * NPAD + ro, RPT)])

    return scatter_k(msg, dst, z2d, z1d, ones_c)


def _tc_final(pA, pB, cA, cB, xpad, maskf, act, root, bias,
              M1a, M1b, mb1, M2, mb2, M3, mb3):
    T = 1280
    G = NPAD // T

    def body(p0, p1, p2, p3, c0, c1, c2, c3, xr, mr, ar, root_r, bias_r,
             m1a, m1b, mb1_r, m2, mb2_r, m3, mb3_r, yr):
        cnt = jnp.maximum(c0[...] + c1[...] + c2[...] + c3[...], 1.0)
        agg = (p0[...] + p1[...] + p2[...] + p3[...]) / cnt
        out = agg + jnp.dot(xr[...], root_r[...],
                            preferred_element_type=jnp.float32) + bias_r[...]
        out = out * mr[...]
        v1 = (jnp.dot(out, m1a[...], preferred_element_type=jnp.float32)
              + ar[...] * m1b[...] + mb1_r[...])
        z1 = jnp.where(v1 > 0, v1, jnp.exp(jnp.minimum(v1, 0.0)) - 1.0)
        v2 = jnp.dot(z1, m2[...], preferred_element_type=jnp.float32) + mb2_r[...]
        z2 = jnp.where(v2 > 0, v2, jnp.exp(jnp.minimum(v2, 0.0)) - 1.0)
        yr[...] = jnp.dot(z2, m3[...], preferred_element_type=jnp.float32) + mb3_r[...]

    part_spec = [pl.BlockSpec((T, OC), lambda i: (i, 0)),
                 pl.BlockSpec((T, OC), lambda i: (i + G, 0))]
    cnt_spec = [pl.BlockSpec((T, 1), lambda i: (i, 0)),
                pl.BlockSpec((T, 1), lambda i: (i + G, 0))]
    return pl.pallas_call(
        body,
        grid=(G,),
        in_specs=part_spec + part_spec + cnt_spec + cnt_spec + [
            pl.BlockSpec((T, SD), lambda i: (i, 0)),
            pl.BlockSpec((T, 1), lambda i: (i, 0)),
            pl.BlockSpec((T, 1), lambda i: (i, 0)),
            pl.BlockSpec((SD, OC), lambda i: (0, 0)),
            pl.BlockSpec((1, OC), lambda i: (0, 0)),
            pl.BlockSpec((OC, 64), lambda i: (0, 0)),
            pl.BlockSpec((1, 64), lambda i: (0, 0)),
            pl.BlockSpec((1, 64), lambda i: (0, 0)),
            pl.BlockSpec((64, 64), lambda i: (0, 0)),
            pl.BlockSpec((1, 64), lambda i: (0, 0)),
            pl.BlockSpec((64, 1), lambda i: (0, 0)),
            pl.BlockSpec((1, 1), lambda i: (0, 0)),
        ],
        out_specs=pl.BlockSpec((T, 1), lambda i: (i, 0)),
        out_shape=jax.ShapeDtypeStruct((NPAD, 1), jnp.float32),
    )(pA, pA, pB, pB, cA, cA, cB, cB, xpad, maskf, act, root, bias,
      M1a, M1b, mb1, M2, mb2, M3, mb3)


def kernel(x, edge_index, edge_attr, mask, batch, action,
           W1, b1, W2, b2, root, bias, M1, mb1, M2, mb2, M3, mb3):
    src = edge_index[0].astype(jnp.int32)
    dst = edge_index[1].astype(jnp.int32)
    eav = (edge_attr.T.reshape(2, 8, 2500, 128)
           .transpose(0, 2, 1, 3).reshape(E * ED))

    z2d = jnp.zeros((NPAD, OC), jnp.float32)
    z1d = jnp.zeros((NPAD,), jnp.float32)
    ones_c = jnp.ones((CH,), jnp.float32)

    parts, cnts = [], []
    for h in range(2):
        xj, eapk_flat = _sc_front(x, src, eav, h)
        msg_pk = _tc_msg(eapk_flat.reshape(EH // 8, 8 * ED),
                         xj.reshape(EH // 8, 8 * SD), W1, b1, W2, b2)
        p, c = _sc_scatter(msg_pk.reshape(EH, OC), dst, z2d, z1d, ones_c, h)
        parts.append(p)
        cnts.append(c)

    pad = NPAD - N
    xpad = jnp.pad(x, ((0, pad), (0, 0)))
    maskf = jnp.pad(mask.astype(jnp.float32), (0, pad)).reshape(NPAD, 1)
    act = jnp.pad(action.astype(jnp.float32), (0, pad)).reshape(NPAD, 1)

    y = _tc_final(parts[0], parts[1],
                  cnts[0].reshape(NC * NPAD, 1), cnts[1].reshape(NC * NPAD, 1),
                  xpad, maskf, act, root, bias.reshape(1, OC),
                  M1[:OC], M1[OC:OC + 1], mb1.reshape(1, 64),
                  M2, mb2.reshape(1, 64), M3, mb3.reshape(1, 1))
    return y[:N]
